# Initial kernel scaffold; baseline (speedup 1.0000x reference)
import jax
import jax.numpy as jnp
from jax.experimental import pallas as pl


def kernel(text_embed, label_embed, target):
    raise NotImplementedError("write your pallas kernel here")


# speedup vs baseline: 69.4986x; 69.4986x over previous
import jax
import jax.numpy as jnp
from jax.experimental import pallas as pl
from jax.experimental.pallas import tpu as pltpu

_HALF = 128
_MARGINS = (0.2, 0.4)


def _loss_body(t_ref, l_ref, tgt_ref, out_ref):
    t = t_ref[...]
    lbl = l_ref[...]
    tn = t / jnp.maximum(jnp.sqrt(jnp.sum(t * t, axis=-1, keepdims=True)), 1e-12)
    ln = lbl / jnp.maximum(jnp.sqrt(jnp.sum(lbl * lbl, axis=-1, keepdims=True)), 1e-12)
    sim = jax.lax.dot_general(
        tn, ln, (((1,), (1,)), ((), ())),
        preferred_element_type=jnp.float32,
        precision=jax.lax.Precision.HIGHEST,
    )
    tgt = tgt_ref[...]

    def half(sub, tv, margin):
        m = jnp.max(sub, axis=1, keepdims=True)
        col = jax.lax.broadcasted_iota(jnp.int32, sub.shape, 1)
        fb = jnp.where(col == 0, sub[:, 1:2], sub[:, 0:1])
        negv = jnp.where(sub < m, m, fb)
        per = jnp.maximum(negv - sub + margin, 0.0)
        vm = jnp.where(tv != 0, 1.0, 0.0)
        return jnp.sum(per * vm), jnp.sum(vm)

    s1, c1 = half(sim[:, :_HALF], tgt[:, :_HALF], _MARGINS[0])
    s2, c2 = half(sim[:, _HALF:], tgt[:, _HALF:], _MARGINS[1])
    loss = s1 / c1 + jnp.where(c2 >= 3.0, s2 / jnp.maximum(c2, 1.0), 0.0)
    out_ref[0, 0] = loss


def kernel(text_embed, label_embed, target):
    tgt = target.astype(jnp.int32)
    out = pl.pallas_call(
        _loss_body,
        out_shape=jax.ShapeDtypeStruct((1, 1), jnp.float32),
        out_specs=pl.BlockSpec(memory_space=pltpu.SMEM),
    )(text_embed, label_embed, tgt)
    return out[0, 0]
